# baseline (device time: 218217 ns/iter reference)
import jax
import jax.numpy as jnp
from jax import lax
from jax.experimental import pallas as pl
from jax.experimental.pallas import tpu as pltpu

N_DEV = 16
N_STEPS = 2 * (N_DEV - 1)

_MESH_COORDS = [(x, y, z) for z in range(4) for (x, y) in ((0, 0), (1, 0), (1, 1), (0, 1))]
_HAM = [(0, 0, 0), (0, 1, 0), (0, 1, 1), (0, 0, 1),
        (0, 0, 2), (0, 1, 2), (0, 1, 3), (0, 0, 3),
        (1, 0, 3), (1, 1, 3), (1, 1, 2), (1, 0, 2),
        (1, 0, 1), (1, 1, 1), (1, 1, 0), (1, 0, 0)]
_PI = [_MESH_COORDS.index(c) for c in _HAM]
_RING = [0] * N_DEV
_SUCC = [0] * N_DEV
_PRED = [0] * N_DEV
for _r, _m in enumerate(_PI):
    _RING[_m] = _r
    _SUCC[_m] = _PI[(_r + 1) % N_DEV]
    _PRED[_m] = _PI[(_r - 1) % N_DEV]

_STREAMS = [
    (0, 48, True), (768, 48, True), (1536, 32, True),
    (2048, 48, False), (2816, 48, False), (3584, 32, False),
]
_GROUPS = ((0, 3), (1, 4), (2, 5))


def kernel(x, w_mat, scale_x, scale_w):
    m, _ = x.shape
    _, n = w_mat.shape
    n_str = len(_STREAMS)

    ring_tab = jnp.array(_RING, dtype=jnp.int32)
    succ_tab = jnp.array(_SUCC, dtype=jnp.int32)
    pred_tab = jnp.array(_PRED, dtype=jnp.int32)

    def body(x_ref, w_ref, sx_ref, sw_ref, ring_ref, succ_ref, pred_ref,
             out_ref, *scr):
        me = lax.axis_index("i")
        r = ring_ref[me]
        nxt = succ_ref[me]
        prv = pred_ref[me]

        s = sx_ref[0] * sw_ref[0]

        comms = scr[0:n_str]
        stages = scr[n_str:2 * n_str]
        ssems = scr[2 * n_str:3 * n_str]
        rsems = scr[3 * n_str:4 * n_str]
        creds = scr[4 * n_str:5 * n_str]

        def indices(k, step):
            fwd = _STREAMS[k][2]
            if step < N_DEV - 1:
                if fwd:
                    send_i = lax.rem(r - step + N_DEV, N_DEV)
                    recv_i = lax.rem(r - step - 1 + N_DEV, N_DEV)
                else:
                    send_i = lax.rem(r + step, N_DEV)
                    recv_i = lax.rem(r + step + 1, N_DEV)
            else:
                t = step - (N_DEV - 1)
                if fwd:
                    send_i = lax.rem(r + 1 - t + N_DEV, N_DEV)
                    recv_i = lax.rem(r - t + N_DEV, N_DEV)
                else:
                    send_i = lax.rem(r - 1 + t + N_DEV, N_DEV)
                    recv_i = lax.rem(r + t, N_DEV)
            return send_i, recv_i

        def row_slice(k, idx):
            row0, ch, _ = _STREAMS[k]
            return pl.ds(row0 + idx * ch, ch)

        def start_step(k, step):
            slot = step % 2
            fwd = _STREAMS[k][2]
            if step == 0:
                send_i, _ = indices(k, 0)
                stages[k][slot] = out_ref[row_slice(k, send_i), :].astype(jnp.bfloat16)
            if step >= 2:
                pl.semaphore_wait(creds[k].at[slot], 1)
            rdma = pltpu.make_async_remote_copy(
                src_ref=stages[k].at[slot], dst_ref=comms[k].at[slot],
                send_sem=ssems[k].at[slot], recv_sem=rsems[k].at[slot],
                device_id=(nxt,) if fwd else (prv,),
                device_id_type=pl.DeviceIdType.MESH,
            )
            rdma.start()
            return rdma

        def finish_step(k, step, rdma):
            rdma.wait()
            slot = step % 2
            nslot = (step + 1) % 2
            fwd = _STREAMS[k][2]
            _, recv_i = indices(k, step)
            tgt = row_slice(k, recv_i)
            arr = comms[k][slot]
            if step < N_DEV - 2:
                stages[k][nslot] = (
                    arr.astype(jnp.float32) + out_ref[tgt, :]
                ).astype(jnp.bfloat16)
            elif step == N_DEV - 2:
                acc = arr.astype(jnp.float32) + out_ref[tgt, :]
                out_ref[tgt, :] = acc
                stages[k][nslot] = acc.astype(jnp.bfloat16)
            elif step < N_STEPS - 1:
                out_ref[tgt, :] = arr.astype(jnp.float32)
                stages[k][nslot] = arr
            else:
                out_ref[tgt, :] = arr.astype(jnp.float32)
            if step < N_STEPS - 2:
                pl.semaphore_signal(
                    creds[k].at[slot], inc=1,
                    device_id=(prv,) if fwd else (nxt,),
                    device_id_type=pl.DeviceIdType.MESH,
                )

        wb = w_ref[...].astype(jnp.bfloat16)
        for k in range(n_str):
            send_i, _ = indices(k, 0)
            rows = row_slice(k, send_i)
            out_ref[rows, :] = (
                jnp.dot(x_ref[rows, :].astype(jnp.bfloat16), wb,
                        preferred_element_type=jnp.float32) * s
            )
        pend = [None] * n_str
        for g in _GROUPS:
            for k in g:
                pend[k] = start_step(k, 0)
        xb = x_ref[...].astype(jnp.bfloat16)
        out_ref[...] = jnp.dot(xb, wb, preferred_element_type=jnp.float32) * s

        for step in range(N_STEPS):
            for g in _GROUPS:
                for k in g:
                    finish_step(k, step, pend[k])
                if step + 1 < N_STEPS:
                    for k in g:
                        pend[k] = start_step(k, step + 1)

    comm_shapes = [pltpu.VMEM((2, ch, n), jnp.bfloat16) for _, ch, _ in _STREAMS]
    return pl.pallas_call(
        body,
        out_shape=jax.ShapeDtypeStruct((m, n), jnp.float32),
        in_specs=[
            pl.BlockSpec(memory_space=pltpu.VMEM),
            pl.BlockSpec(memory_space=pltpu.VMEM),
            pl.BlockSpec(memory_space=pltpu.SMEM),
            pl.BlockSpec(memory_space=pltpu.SMEM),
            pl.BlockSpec(memory_space=pltpu.SMEM),
            pl.BlockSpec(memory_space=pltpu.SMEM),
            pl.BlockSpec(memory_space=pltpu.SMEM),
        ],
        out_specs=pl.BlockSpec(memory_space=pltpu.VMEM),
        scratch_shapes=(
            comm_shapes
            + comm_shapes
            + [pltpu.SemaphoreType.DMA((2,))] * n_str
            + [pltpu.SemaphoreType.DMA((2,))] * n_str
            + [pltpu.SemaphoreType.REGULAR((2,))] * n_str
        ),
        compiler_params=pltpu.CompilerParams(
            vmem_limit_bytes=100 * 1024 * 1024,
        ),
    )(x, w_mat, scale_x, scale_w, ring_tab, succ_tab, pred_tab)


# device time: 217707 ns/iter; 1.0023x vs baseline; 1.0023x over previous
import jax
import jax.numpy as jnp
from jax import lax
from jax.experimental import pallas as pl
from jax.experimental.pallas import tpu as pltpu

N_DEV = 16
N_STEPS = 2 * (N_DEV - 1)

_MESH_COORDS = [(x, y, z) for z in range(4) for (x, y) in ((0, 0), (1, 0), (1, 1), (0, 1))]
_HAM = [(0, 0, 0), (0, 1, 0), (0, 1, 1), (0, 0, 1),
        (0, 0, 2), (0, 1, 2), (0, 1, 3), (0, 0, 3),
        (1, 0, 3), (1, 1, 3), (1, 1, 2), (1, 0, 2),
        (1, 0, 1), (1, 1, 1), (1, 1, 0), (1, 0, 0)]
_PI = [_MESH_COORDS.index(c) for c in _HAM]
_RING = [0] * N_DEV
_SUCC = [0] * N_DEV
_PRED = [0] * N_DEV
for _r, _m in enumerate(_PI):
    _RING[_m] = _r
    _SUCC[_m] = _PI[(_r + 1) % N_DEV]
    _PRED[_m] = _PI[(_r - 1) % N_DEV]


def kernel(x, w_mat, scale_x, scale_w):
    m, _ = x.shape
    _, n = w_mat.shape
    q = m // 4
    ch = q // N_DEV

    ring_tab = jnp.array(_RING, dtype=jnp.int32)
    succ_tab = jnp.array(_SUCC, dtype=jnp.int32)
    pred_tab = jnp.array(_PRED, dtype=jnp.int32)

    def body(x_ref, w_ref, sx_ref, sw_ref, ring_ref, succ_ref, pred_ref,
             out_ref, *scr):
        me = lax.axis_index("i")
        r = ring_ref[me]
        nxt = succ_ref[me]
        prv = pred_ref[me]

        s = sx_ref[0] * sw_ref[0]

        comms = scr[0:4]
        stages = scr[4:8]
        ssems = scr[8:12]
        rsems = scr[12:16]
        creds = scr[16:20]
        row0s = [0, q, 2 * q, 3 * q]
        fwds = [True, True, False, False]

        def indices(k, step):
            fwd = fwds[k]
            if step < N_DEV - 1:
                if fwd:
                    send_i = lax.rem(r - step + N_DEV, N_DEV)
                    recv_i = lax.rem(r - step - 1 + N_DEV, N_DEV)
                else:
                    send_i = lax.rem(r + step, N_DEV)
                    recv_i = lax.rem(r + step + 1, N_DEV)
            else:
                t = step - (N_DEV - 1)
                if fwd:
                    send_i = lax.rem(r + 1 - t + N_DEV, N_DEV)
                    recv_i = lax.rem(r - t + N_DEV, N_DEV)
                else:
                    send_i = lax.rem(r - 1 + t + N_DEV, N_DEV)
                    recv_i = lax.rem(r + t, N_DEV)
            return send_i, recv_i

        def start_step(k, step):
            slot = step % 2
            if step == 0:
                send_i, _ = indices(k, 0)
                stages[k][slot] = out_ref[
                    pl.ds(row0s[k] + send_i * ch, ch), :
                ].astype(jnp.bfloat16)
            if step >= 2:
                pl.semaphore_wait(creds[k].at[slot], 1)
            rdma = pltpu.make_async_remote_copy(
                src_ref=stages[k].at[slot], dst_ref=comms[k].at[slot],
                send_sem=ssems[k].at[slot], recv_sem=rsems[k].at[slot],
                device_id=(nxt,) if fwds[k] else (prv,),
                device_id_type=pl.DeviceIdType.MESH,
            )
            rdma.start()
            return rdma

        def finish_step(k, step, rdma):
            rdma.wait()
            slot = step % 2
            nslot = (step + 1) % 2
            _, recv_i = indices(k, step)
            tgt = pl.ds(row0s[k] + recv_i * ch, ch)
            arr = comms[k][slot]
            if step < N_DEV - 2:
                stages[k][nslot] = (
                    arr.astype(jnp.float32) + out_ref[tgt, :]
                ).astype(jnp.bfloat16)
            elif step == N_DEV - 2:
                acc = arr.astype(jnp.float32) + out_ref[tgt, :]
                out_ref[tgt, :] = acc
                stages[k][nslot] = acc.astype(jnp.bfloat16)
            elif step < N_STEPS - 1:
                out_ref[tgt, :] = arr.astype(jnp.float32)
                stages[k][nslot] = arr
            else:
                out_ref[tgt, :] = arr.astype(jnp.float32)
            if step < N_STEPS - 2:
                pl.semaphore_signal(
                    creds[k].at[slot], inc=1,
                    device_id=(prv,) if fwds[k] else (nxt,),
                    device_id_type=pl.DeviceIdType.MESH,
                )

        groups = ((0, 2), (1, 3))

        wb = w_ref[...].astype(jnp.bfloat16)

        def compute_slab(k, j):
            rows = pl.ds(row0s[k] + j * ch, ch)
            out_ref[rows, :] = (
                jnp.dot(x_ref[rows, :].astype(jnp.bfloat16), wb,
                        preferred_element_type=jnp.float32) * s
            )

        for k in range(4):
            send_i, _ = indices(k, 0)
            compute_slab(k, send_i)
        pend = [None] * 4
        for g in groups:
            for k in g:
                pend[k] = start_step(k, 0)

        for step in range(N_STEPS):
            if step < N_DEV - 1:
                for k in range(4):
                    _, recv_i = indices(k, step)
                    compute_slab(k, recv_i)
            for g in groups:
                for k in g:
                    finish_step(k, step, pend[k])
                if step + 1 < N_STEPS:
                    for k in g:
                        pend[k] = start_step(k, step + 1)

    return pl.pallas_call(
        body,
        out_shape=jax.ShapeDtypeStruct((m, n), jnp.float32),
        in_specs=[
            pl.BlockSpec(memory_space=pltpu.VMEM),
            pl.BlockSpec(memory_space=pltpu.VMEM),
            pl.BlockSpec(memory_space=pltpu.SMEM),
            pl.BlockSpec(memory_space=pltpu.SMEM),
            pl.BlockSpec(memory_space=pltpu.SMEM),
            pl.BlockSpec(memory_space=pltpu.SMEM),
            pl.BlockSpec(memory_space=pltpu.SMEM),
        ],
        out_specs=pl.BlockSpec(memory_space=pltpu.VMEM),
        scratch_shapes=(
            [pltpu.VMEM((2, ch, n), jnp.bfloat16)] * 4
            + [pltpu.VMEM((2, ch, n), jnp.bfloat16)] * 4
            + [pltpu.SemaphoreType.DMA((2,))] * 4
            + [pltpu.SemaphoreType.DMA((2,))] * 4
            + [pltpu.SemaphoreType.REGULAR((2,))] * 4
        ),
        compiler_params=pltpu.CompilerParams(
            vmem_limit_bytes=100 * 1024 * 1024,
        ),
    )(x, w_mat, scale_x, scale_w, ring_tab, succ_tab, pred_tab)


# device time: 211033 ns/iter; 1.0340x vs baseline; 1.0316x over previous
import jax
import jax.numpy as jnp
from jax import lax
from jax.experimental import pallas as pl
from jax.experimental.pallas import tpu as pltpu

N_DEV = 16
N_STEPS = 2 * (N_DEV - 1)

_MESH_COORDS = [(x, y, z) for z in range(4) for (x, y) in ((0, 0), (1, 0), (1, 1), (0, 1))]
_HAM = [(0, 0, 0), (0, 1, 0), (0, 1, 1), (0, 0, 1),
        (0, 0, 2), (0, 1, 2), (0, 1, 3), (0, 0, 3),
        (1, 0, 3), (1, 1, 3), (1, 1, 2), (1, 0, 2),
        (1, 0, 1), (1, 1, 1), (1, 1, 0), (1, 0, 0)]
_PI = [_MESH_COORDS.index(c) for c in _HAM]
_RING = [0] * N_DEV
_SUCC = [0] * N_DEV
_PRED = [0] * N_DEV
for _r, _m in enumerate(_PI):
    _RING[_m] = _r
    _SUCC[_m] = _PI[(_r + 1) % N_DEV]
    _PRED[_m] = _PI[(_r - 1) % N_DEV]


def kernel(x, w_mat, scale_x, scale_w):
    m, _ = x.shape
    _, n = w_mat.shape
    q = m // 4
    ch = q // N_DEV

    ring_tab = jnp.array(_RING, dtype=jnp.int32)
    succ_tab = jnp.array(_SUCC, dtype=jnp.int32)
    pred_tab = jnp.array(_PRED, dtype=jnp.int32)

    def body(x_ref, w_ref, sx_ref, sw_ref, ring_ref, succ_ref, pred_ref,
             out_ref, *scr):
        me = lax.axis_index("i")
        r = ring_ref[me]
        nxt = succ_ref[me]
        prv = pred_ref[me]

        s = sx_ref[0] * sw_ref[0]

        barrier_sem = pltpu.get_barrier_semaphore()
        for nbr in (nxt, prv):
            pl.semaphore_signal(
                barrier_sem, inc=1,
                device_id=(nbr,), device_id_type=pl.DeviceIdType.MESH,
            )
        pl.semaphore_wait(barrier_sem, 2)

        comms = scr[0:4]
        stages = scr[4:8]
        ssems = scr[8:12]
        rsems = scr[12:16]
        creds = scr[16:20]
        row0s = [0, q, 2 * q, 3 * q]
        fwds = [True, True, False, False]

        def indices(k, step):
            fwd = fwds[k]
            if step < N_DEV - 1:
                if fwd:
                    send_i = lax.rem(r - step + N_DEV, N_DEV)
                    recv_i = lax.rem(r - step - 1 + N_DEV, N_DEV)
                else:
                    send_i = lax.rem(r + step, N_DEV)
                    recv_i = lax.rem(r + step + 1, N_DEV)
            else:
                t = step - (N_DEV - 1)
                if fwd:
                    send_i = lax.rem(r + 1 - t + N_DEV, N_DEV)
                    recv_i = lax.rem(r - t + N_DEV, N_DEV)
                else:
                    send_i = lax.rem(r - 1 + t + N_DEV, N_DEV)
                    recv_i = lax.rem(r + t, N_DEV)
            return send_i, recv_i

        def start_step(k, step):
            slot = step % 2
            if step == 0:
                send_i, _ = indices(k, 0)
                stages[k][slot] = out_ref[
                    pl.ds(row0s[k] + send_i * ch, ch), :
                ].astype(jnp.bfloat16)
            if step >= 2:
                pl.semaphore_wait(creds[k].at[slot], 1)
            rdma = pltpu.make_async_remote_copy(
                src_ref=stages[k].at[slot], dst_ref=comms[k].at[slot],
                send_sem=ssems[k].at[slot], recv_sem=rsems[k].at[slot],
                device_id=(nxt,) if fwds[k] else (prv,),
                device_id_type=pl.DeviceIdType.MESH,
            )
            rdma.start()
            return rdma

        def finish_step(k, step, rdma):
            rdma.wait()
            slot = step % 2
            nslot = (step + 1) % 2
            _, recv_i = indices(k, step)
            tgt = pl.ds(row0s[k] + recv_i * ch, ch)
            arr = comms[k][slot]
            if step < N_DEV - 2:
                stages[k][nslot] = (
                    arr.astype(jnp.float32) + out_ref[tgt, :]
                ).astype(jnp.bfloat16)
            elif step == N_DEV - 2:
                acc = arr.astype(jnp.float32) + out_ref[tgt, :]
                out_ref[tgt, :] = acc
                stages[k][nslot] = acc.astype(jnp.bfloat16)
            elif step < N_STEPS - 1:
                out_ref[tgt, :] = arr.astype(jnp.float32)
                stages[k][nslot] = arr
            else:
                out_ref[tgt, :] = arr.astype(jnp.float32)
            if step < N_STEPS - 2:
                pl.semaphore_signal(
                    creds[k].at[slot], inc=1,
                    device_id=(prv,) if fwds[k] else (nxt,),
                    device_id_type=pl.DeviceIdType.MESH,
                )

        groups = ((0, 2), (1, 3))

        wb = w_ref[...].astype(jnp.bfloat16)

        def compute_slab(k, j):
            rows = pl.ds(row0s[k] + j * ch, ch)
            out_ref[rows, :] = (
                jnp.dot(x_ref[rows, :].astype(jnp.bfloat16), wb,
                        preferred_element_type=jnp.float32) * s
            )

        for k in range(4):
            send_i, _ = indices(k, 0)
            compute_slab(k, send_i)
        pend = [None] * 4
        for g in groups:
            for k in g:
                pend[k] = start_step(k, 0)

        for step in range(N_STEPS):
            if step < N_DEV - 1:
                for k in range(4):
                    _, recv_i = indices(k, step)
                    compute_slab(k, recv_i)
            for g in groups:
                for k in g:
                    finish_step(k, step, pend[k])
                if step + 1 < N_STEPS:
                    for k in g:
                        pend[k] = start_step(k, step + 1)

    return pl.pallas_call(
        body,
        out_shape=jax.ShapeDtypeStruct((m, n), jnp.float32),
        in_specs=[
            pl.BlockSpec(memory_space=pltpu.VMEM),
            pl.BlockSpec(memory_space=pltpu.VMEM),
            pl.BlockSpec(memory_space=pltpu.SMEM),
            pl.BlockSpec(memory_space=pltpu.SMEM),
            pl.BlockSpec(memory_space=pltpu.SMEM),
            pl.BlockSpec(memory_space=pltpu.SMEM),
            pl.BlockSpec(memory_space=pltpu.SMEM),
        ],
        out_specs=pl.BlockSpec(memory_space=pltpu.VMEM),
        scratch_shapes=(
            [pltpu.VMEM((2, ch, n), jnp.bfloat16)] * 4
            + [pltpu.VMEM((2, ch, n), jnp.bfloat16)] * 4
            + [pltpu.SemaphoreType.DMA((2,))] * 4
            + [pltpu.SemaphoreType.DMA((2,))] * 4
            + [pltpu.SemaphoreType.REGULAR((2,))] * 4
        ),
        compiler_params=pltpu.CompilerParams(
            vmem_limit_bytes=100 * 1024 * 1024,
            collective_id=0,
        ),
    )(x, w_mat, scale_x, scale_w, ring_tab, succ_tab, pred_tab)
